# baseline (speedup 1.0000x reference)
import functools

import jax
import jax.numpy as jnp
from jax import lax
from jax.experimental import pallas as pl
from jax.experimental.pallas import tpu as pltpu
from jax.experimental.pallas import tpu_sc as plsc

NC = 2
NS = 16
NW = NC * NS
BLK = 128
NBUF = 4
L = 16


def _sc_embedding_lookup(xt, weight, n_pos, batch):
    n_rows, d = weight.shape
    bpw = batch // NW
    ncs = bpw // BLK
    nblocks = n_pos * ncs
    dg, di = d // 8, 8

    mesh = plsc.VectorSubcoreMesh(core_axis_name="c", subcore_axis_name="s")

    @functools.partial(
        pl.kernel,
        out_type=jax.ShapeDtypeStruct((n_pos, dg, batch // BLK, di, BLK),
                                      jnp.float32),
        mesh=mesh,
        scratch_types=[
            pltpu.VMEM((n_pos, bpw), jnp.int32),
            *([pltpu.VMEM((BLK, d), jnp.float32)] * NBUF),
            *([pltpu.VMEM((dg, di, BLK), jnp.float32)] * NBUF),
            *([pltpu.SemaphoreType.DMA] * NBUF),
            *([pltpu.SemaphoreType.DMA] * NBUF),
        ],
        compiler_params=pltpu.CompilerParams(
            use_tc_tiling_on_sc=False, needs_layout_passes=False),
    )
    def k(xt_hbm, w_hbm, out_hbm, xt_v, *bufs_and_sems):
        g = bufs_and_sems[:NBUF]
        t = bufs_and_sems[NBUF:2 * NBUF]
        sem_g = bufs_and_sems[2 * NBUF:3 * NBUF]
        sem_o = bufs_and_sems[3 * NBUF:4 * NBUF]

        wid = lax.axis_index("s") * NC + lax.axis_index("c")
        c0 = wid * bpw
        pltpu.sync_copy(xt_hbm.at[:, pl.ds(c0, bpw)], xt_v)

        jvecs = [lax.iota(jnp.int32, L) + j0 * L for j0 in range(BLK // L)]

        def start_gather(blkid, b):
            s = blkid // ncs
            cs = blkid % ncs
            pltpu.async_copy(
                w_hbm.at[xt_v.at[s, pl.ds(cs * BLK, BLK)]], g[b], sem_g[b])

        def wait_gather(b):
            pltpu.make_async_copy(
                w_hbm.at[xt_v.at[0, pl.ds(0, BLK)]], g[b], sem_g[b]).wait()

        def transpose_block(b):
            def dii_body(dii, carry):
                dii_splat = jnp.full((L,), 0, jnp.int32) + dii
                for dgi in range(dg):
                    dsplat = dii_splat + (dgi * di)
                    for j0 in range(BLK // L):
                        vals = plsc.load_gather(g[b], [jvecs[j0], dsplat])
                        t[b][dgi, dii, pl.ds(j0 * L, L)] = vals
                return carry

            lax.fori_loop(0, di, dii_body, 0)

        def out_slice(blkid):
            s = blkid // ncs
            cs = blkid % ncs
            return out_hbm.at[s, :, (c0 // BLK) + cs]

        def start_out(blkid, b):
            pltpu.async_copy(t[b], out_slice(blkid), sem_o[b])

        def wait_out(b):
            pltpu.make_async_copy(t[b], out_hbm.at[0, :, 0], sem_o[b]).wait()

        for b in range(NBUF):
            start_gather(b, b)

        ngroups = nblocks // NBUF

        def group_body(grp, carry):
            b0 = grp * NBUF
            for b in range(NBUF):
                wait_gather(b)
                transpose_block(b)
                start_out(b0 + b, b)
            for b in range(NBUF):
                wait_out(b)
                start_gather(lax.rem(b0 + NBUF + b, nblocks), b)
            return carry

        lax.fori_loop(0, ngroups, group_body, 0)

        for b in range(NBUF):
            wait_gather(b)

    return k(xt, weight)


def kernel(x, weight):
    batch, n_pos = x.shape
    d = weight.shape[1]
    xt = jnp.transpose(x).astype(jnp.int32)
    out5 = _sc_embedding_lookup(xt, weight, n_pos, batch)
    out = jnp.transpose(out5, (2, 4, 0, 1, 3)).reshape(batch, n_pos, d)
    return out


# speedup vs baseline: 1.0128x; 1.0128x over previous
import functools

import jax
import jax.numpy as jnp
from jax import lax
from jax.experimental import pallas as pl
from jax.experimental.pallas import tpu as pltpu
from jax.experimental.pallas import tpu_sc as plsc

NC = 2
NS = 16
NW = NC * NS
BLK = 128
NBUF = 8
L = 16


def _sc_embedding_lookup(xt, weight, n_pos, batch):
    n_rows, d = weight.shape
    bpw = batch // NW
    ncs = bpw // BLK
    nblocks = n_pos * ncs
    dg, di = d // 8, 8

    mesh = plsc.VectorSubcoreMesh(core_axis_name="c", subcore_axis_name="s")

    @functools.partial(
        pl.kernel,
        out_type=jax.ShapeDtypeStruct((n_pos, dg, batch // BLK, di, BLK),
                                      jnp.float32),
        mesh=mesh,
        scratch_types=[
            pltpu.VMEM((n_pos, bpw), jnp.int32),
            *([pltpu.VMEM((BLK, d), jnp.float32)] * NBUF),
            *([pltpu.VMEM((dg, di, BLK), jnp.float32)] * NBUF),
            *([pltpu.SemaphoreType.DMA] * NBUF),
            *([pltpu.SemaphoreType.DMA] * NBUF),
        ],
        compiler_params=pltpu.CompilerParams(
            use_tc_tiling_on_sc=False, needs_layout_passes=False),
    )
    def k(xt_hbm, w_hbm, out_hbm, xt_v, *bufs_and_sems):
        g = bufs_and_sems[:NBUF]
        t = bufs_and_sems[NBUF:2 * NBUF]
        sem_g = bufs_and_sems[2 * NBUF:3 * NBUF]
        sem_o = bufs_and_sems[3 * NBUF:4 * NBUF]

        wid = lax.axis_index("s") * NC + lax.axis_index("c")
        c0 = wid * bpw
        pltpu.sync_copy(xt_hbm.at[:, pl.ds(c0, bpw)], xt_v)

        jvecs = [lax.iota(jnp.int32, L) + j0 * L for j0 in range(BLK // L)]

        def start_gather(blkid, b):
            s = blkid // ncs
            cs = blkid % ncs
            pltpu.async_copy(
                w_hbm.at[xt_v.at[s, pl.ds(cs * BLK, BLK)]], g[b], sem_g[b])

        def wait_gather(b):
            pltpu.make_async_copy(
                w_hbm.at[xt_v.at[0, pl.ds(0, BLK)]], g[b], sem_g[b]).wait()

        def transpose_block(b):
            def dii_body(dii, carry):
                dii_splat = jnp.full((L,), 0, jnp.int32) + dii
                for dgi in range(dg):
                    dsplat = dii_splat + (dgi * di)
                    for j0 in range(BLK // L):
                        vals = plsc.load_gather(g[b], [jvecs[j0], dsplat])
                        t[b][dgi, dii, pl.ds(j0 * L, L)] = vals
                return carry

            lax.fori_loop(0, di, dii_body, 0)

        def out_slice(blkid):
            s = blkid // ncs
            cs = blkid % ncs
            return out_hbm.at[s, :, (c0 // BLK) + cs]

        def start_out(blkid, b):
            pltpu.async_copy(t[b], out_slice(blkid), sem_o[b])

        def wait_out(b):
            pltpu.make_async_copy(t[b], out_hbm.at[0, :, 0], sem_o[b]).wait()

        for b in range(NBUF):
            start_gather(b, b)

        ngroups = nblocks // NBUF

        def group_body(grp, carry):
            b0 = grp * NBUF
            for b in range(NBUF):
                wait_gather(b)
                transpose_block(b)
                start_out(b0 + b, b)
            for b in range(NBUF):
                wait_out(b)
                start_gather(lax.rem(b0 + NBUF + b, nblocks), b)
            return carry

        lax.fori_loop(0, ngroups, group_body, 0)

        for b in range(NBUF):
            wait_gather(b)

    return k(xt, weight)


def kernel(x, weight):
    batch, n_pos = x.shape
    d = weight.shape[1]
    xt = jnp.transpose(x).astype(jnp.int32)
    out5 = _sc_embedding_lookup(xt, weight, n_pos, batch)
    out = jnp.transpose(out5, (2, 4, 0, 1, 3)).reshape(batch, n_pos, d)
    return out


# speedup vs baseline: 1.5937x; 1.5735x over previous
import functools

import jax
import jax.numpy as jnp
from jax import lax
from jax.experimental import pallas as pl
from jax.experimental.pallas import tpu as pltpu
from jax.experimental.pallas import tpu_sc as plsc

NC = 2
NS = 16
NW = NC * NS
BLK = 128
NBUF = 8
L = 16


def _sc_embedding_lookup(xt, weight, n_pos, batch):
    n_rows, d = weight.shape
    bpw = batch // NW
    ncs = bpw // BLK
    nblocks = n_pos * ncs
    dg, di = d // 8, 8

    mesh = plsc.VectorSubcoreMesh(core_axis_name="c", subcore_axis_name="s")

    @functools.partial(
        pl.kernel,
        out_type=jax.ShapeDtypeStruct((n_pos, dg, batch // BLK, di, BLK),
                                      jnp.float32),
        mesh=mesh,
        scratch_types=[
            pltpu.VMEM((n_pos, bpw), jnp.int32),
            *([pltpu.VMEM((BLK, d), jnp.float32)] * NBUF),
            *([pltpu.VMEM((dg, di, BLK + 1), jnp.float32)] * NBUF),
            *([pltpu.SemaphoreType.DMA] * NBUF),
            *([pltpu.SemaphoreType.DMA] * NBUF),
        ],
        compiler_params=pltpu.CompilerParams(
            use_tc_tiling_on_sc=False, needs_layout_passes=False),
    )
    def k(xt_hbm, w_hbm, out_hbm, xt_v, *bufs_and_sems):
        g = bufs_and_sems[:NBUF]
        t = bufs_and_sems[NBUF:2 * NBUF]
        sem_g = bufs_and_sems[2 * NBUF:3 * NBUF]
        sem_o = bufs_and_sems[3 * NBUF:4 * NBUF]

        wid = lax.axis_index("s") * NC + lax.axis_index("c")
        c0 = wid * bpw
        pltpu.sync_copy(xt_hbm.at[:, pl.ds(c0, bpw)], xt_v)

        dlane = lax.iota(jnp.int32, L)
        dg_idx = [(dlane + half * L) // di for half in range(d // L)]
        di_idx = [(dlane + half * L) % di for half in range(d // L)]

        def start_gather(blkid, b):
            s = blkid // ncs
            cs = blkid % ncs
            pltpu.async_copy(
                w_hbm.at[xt_v.at[s, pl.ds(cs * BLK, BLK)]], g[b], sem_g[b])

        def wait_gather(b):
            pltpu.make_async_copy(
                w_hbm.at[xt_v.at[0, pl.ds(0, BLK)]], g[b], sem_g[b]).wait()

        UNROLL_J = 4

        def transpose_block(b):
            def j_body(j0, carry):
                jbase = j0 * UNROLL_J
                for ju in range(UNROLL_J):
                    jsplat = jnp.full((L,), 0, jnp.int32) + (jbase + ju)
                    for half in range(d // L):
                        vals = g[b][jbase + ju, pl.ds(half * L, L)]
                        plsc.store_scatter(
                            t[b], [dg_idx[half], di_idx[half], jsplat], vals)
                return carry

            lax.fori_loop(0, BLK // UNROLL_J, j_body, 0)

        def out_slice(blkid):
            s = blkid // ncs
            cs = blkid % ncs
            return out_hbm.at[s, :, (c0 // BLK) + cs]

        def start_out(blkid, b):
            pltpu.async_copy(
                t[b].at[:, :, pl.ds(0, BLK)], out_slice(blkid), sem_o[b])

        def wait_out(b):
            pltpu.make_async_copy(
                t[b].at[:, :, pl.ds(0, BLK)], out_hbm.at[0, :, 0],
                sem_o[b]).wait()

        for b in range(NBUF):
            start_gather(b, b)

        ngroups = nblocks // NBUF

        def group_body(grp, carry):
            b0 = grp * NBUF
            for b in range(NBUF):
                wait_gather(b)
                transpose_block(b)
                start_out(b0 + b, b)
            for b in range(NBUF):
                wait_out(b)
                start_gather(lax.rem(b0 + NBUF + b, nblocks), b)
            return carry

        lax.fori_loop(0, ngroups, group_body, 0)

        for b in range(NBUF):
            wait_gather(b)

    return k(xt, weight)


def kernel(x, weight):
    batch, n_pos = x.shape
    d = weight.shape[1]
    xt = jnp.transpose(x).astype(jnp.int32)
    out5 = _sc_embedding_lookup(xt, weight, n_pos, batch)
    out = jnp.transpose(out5, (2, 4, 0, 1, 3)).reshape(batch, n_pos, d)
    return out
